# baseline (speedup 1.0000x reference)
import functools

import jax
import jax.numpy as jnp
from jax import lax
from jax.experimental import pallas as pl
from jax.experimental.pallas import tpu as pltpu
from jax.experimental.pallas import tpu_sc as plsc

NSEG = 128
NROWS = 50000
D = 512
LANES = 16
C = 48
G = C // LANES
NC = 2
NS = 16
NW = NC * NS
Q = (-(-NROWS // NW) + 7) // 8 * 8
NWIN = -(-Q // C)


def _tree_sum(vs):
  while len(vs) > 1:
    nxt = [a + b for a, b in zip(vs[::2], vs[1::2])]
    if len(vs) % 2:
      nxt.append(vs[-1])
    vs = nxt
  return vs[0]


def _sc_segment_sums(x, batch_i32):
  mesh = plsc.VectorSubcoreMesh(core_axis_name="c", subcore_axis_name="s")

  @functools.partial(
      pl.kernel,
      mesh=mesh,
      compiler_params=pltpu.CompilerParams(needs_layout_passes=False),
      out_type=[
          jax.ShapeDtypeStruct((NW, NSEG * D), jnp.float32),
          jax.ShapeDtypeStruct((NW, NSEG * LANES), jnp.float32),
      ],
      scratch_types=[
          pltpu.VMEM((C,), jnp.int32),
          pltpu.VMEM((C,), jnp.int32),
          pltpu.VMEM((C, D), jnp.float32),
          pltpu.VMEM((C, D), jnp.float32),
          pltpu.VMEM((NSEG * D,), jnp.float32),
          pltpu.VMEM((NSEG * LANES,), jnp.float32),
          pltpu.SemaphoreType.DMA,
          pltpu.SemaphoreType.DMA,
          pltpu.SemaphoreType.DMA,
          pltpu.SemaphoreType.DMA,
      ],
  )
  def seg_sum(x_hbm, b_hbm, sums_hbm, cnts_hbm,
              idx0, idx1, rows0, rows1, acc_v, cacc_v,
              semi0, semi1, semx0, semx1):
    cid = lax.axis_index("c")
    sid = lax.axis_index("s")
    wid = sid * NC + cid

    zeros = jnp.zeros((LANES,), jnp.float32)
    ones = jnp.ones((LANES,), jnp.float32)
    lane_iota = lax.iota(jnp.int32, LANES)

    def zbody(i, carry):
      for j in range(D // LANES):
        acc_v[pl.ds(i * D + j * LANES, LANES)] = zeros
      cacc_v[pl.ds(i * LANES, LANES)] = zeros
      return carry

    lax.fori_loop(0, NSEG, zbody, 0)

    start = wid * Q
    end = jnp.minimum(start + Q, NROWS)

    idx_b = [idx0, idx1]
    rows_b = [rows0, rows1]
    semi = [semi0, semi1]
    semx = [semx0, semx1]

    def wstart(i):
      return jnp.minimum(start + i * C, NROWS - C)

    def issue(i, b):
      ws = wstart(i)
      pltpu.async_copy(b_hbm.at[pl.ds(ws, C)], idx_b[b], semi[b])
      pltpu.async_copy(x_hbm.at[pl.ds(ws, C)], rows_b[b], semx[b])

    def wait(i, b):
      ws = wstart(i)
      pltpu.make_async_copy(b_hbm.at[pl.ds(ws, C)], idx_b[b], semi[b]).wait()
      pltpu.make_async_copy(x_hbm.at[pl.ds(ws, C)], rows_b[b], semx[b]).wait()

    def process(i, b):
      lo = start + i * C
      ws = wstart(i)
      ib = idx_b[b]
      rb = rows_b[b]

      ids_first = ib[pl.ds(0, LANES)]
      ids_last = ib[pl.ds(C - LANES, LANES)]
      wuni = ((ids_first[0] == ids_last[LANES - 1])
              & (ws >= lo) & (ws + C <= end))

      @pl.when(wuni)
      def _window_uniform():
        for g in range(G):
          ids16 = ib[pl.ds(g * LANES, LANES)]
          plsc.addupdate_scatter(cacc_v, [ids16 * LANES + lane_iota], ones)
        addr = lax.broadcast(ids_first[0] * D, (LANES,)) + lane_iota

        def jbody(j, carry):
          parts = []
          for g in range(G):
            parts.append(_tree_sum(
                [rb[g * LANES + l, pl.ds(j * LANES, LANES)]
                 for l in range(LANES)]))
          plsc.addupdate_scatter(acc_v, [addr + j * LANES], _tree_sum(parts))
          return carry

        lax.fori_loop(0, D // LANES, jbody, 0)

      def gbody(g, carry):
        r0 = ws + g * LANES
        ids16 = ib[pl.ds(g * LANES, LANES)]
        gr = lax.broadcast(r0, (LANES,)) + lane_iota
        vmask = (gr >= lo) & (gr < end)
        plsc.addupdate_scatter(
            cacc_v, [ids16 * LANES + lane_iota], ones, mask=vmask)

        full = (ids16[0] == ids16[LANES - 1]) & (r0 >= lo) & (r0 + LANES <= end)

        @pl.when(full)
        def _fast():
          addr = lax.broadcast(ids16[0] * D, (LANES,)) + lane_iota
          for j in range(D // LANES):
            s = _tree_sum(
                [rb[g * LANES + l, pl.ds(j * LANES, LANES)]
                 for l in range(LANES)])
            plsc.addupdate_scatter(acc_v, [addr + (j * LANES)], s)

        @pl.when(jnp.logical_not(full))
        def _slow():
          idsD = ids16 * D
          for l in range(LANES):
            rl = r0 + l
            inb = (rl >= lo) & (rl < end)
            m = lax.broadcast(inb, (LANES,))
            seg = lax.broadcast(idsD[l], (LANES,)) + lane_iota

            def sjbody(j, carry3, _l=l, _seg=seg, _m=m):
              plsc.addupdate_scatter(
                  acc_v, [_seg + j * LANES],
                  rb[g * LANES + _l, pl.ds(j * LANES, LANES)], mask=_m)
              return carry3

            lax.fori_loop(0, D // LANES, sjbody, 0)

        return carry

      @pl.when(jnp.logical_not(wuni) & (lo < end))
      def _():
        lax.fori_loop(0, G, gbody, 0)

    issue(0, 0)

    def pbody(p, carry):
      w = p * 2
      issue(w + 1, 1)
      wait(w, 0)
      process(w, 0)
      issue(w + 2, 0)
      wait(w + 1, 1)
      process(w + 1, 1)
      return carry

    lax.fori_loop(0, (NWIN - 1) // 2, pbody, 0)
    wait(NWIN - 1, 0)
    process(NWIN - 1, 0)

    pltpu.sync_copy(acc_v, sums_hbm.at[wid])
    pltpu.sync_copy(cacc_v, cnts_hbm.at[wid])

  return seg_sum(x, batch_i32)


def _combine(sums, cnts):
  def body(s_ref, c_ref, o_ref):
    s = jnp.sum(s_ref[...], axis=0)
    c = jnp.sum(c_ref[...], axis=(0, 2))
    o_ref[...] = s / jnp.maximum(c, 1.0)[:, None]

  return pl.pallas_call(
      body,
      out_shape=jax.ShapeDtypeStruct((NSEG, D), jnp.float32),
  )(sums, cnts)


@jax.jit
def kernel(x, batch):
  sums, cnts = _sc_segment_sums(x, batch.astype(jnp.int32))
  sums = sums.reshape(NW, NSEG, D)
  cnts = cnts.reshape(NW, NSEG, LANES)
  return _combine(sums, cnts)


# speedup vs baseline: 1.0564x; 1.0564x over previous
import functools

import jax
import jax.numpy as jnp
from jax import lax
from jax.experimental import pallas as pl
from jax.experimental.pallas import tpu as pltpu
from jax.experimental.pallas import tpu_sc as plsc

NSEG = 128
NROWS = 50000
D = 512
LANES = 16
C = 48
G = C // LANES
NC = 2
NS = 16
NW = NC * NS
Q = (-(-NROWS // NW) + 7) // 8 * 8
NWIN = -(-Q // C)


def _tree_sum(vs):
  while len(vs) > 1:
    nxt = [a + b for a, b in zip(vs[::2], vs[1::2])]
    if len(vs) % 2:
      nxt.append(vs[-1])
    vs = nxt
  return vs[0]


def _sc_segment_sums(x, batch_i32):
  mesh = plsc.VectorSubcoreMesh(core_axis_name="c", subcore_axis_name="s")

  @functools.partial(
      pl.kernel,
      mesh=mesh,
      compiler_params=pltpu.CompilerParams(needs_layout_passes=False),
      out_type=[
          jax.ShapeDtypeStruct((NW, NSEG * D), jnp.float32),
          jax.ShapeDtypeStruct((NW, NSEG * LANES), jnp.float32),
      ],
      scratch_types=[
          pltpu.VMEM((C,), jnp.int32),
          pltpu.VMEM((C,), jnp.int32),
          pltpu.VMEM((C, D), jnp.float32),
          pltpu.VMEM((C, D), jnp.float32),
          pltpu.VMEM((NSEG * D,), jnp.float32),
          pltpu.VMEM((NSEG * LANES,), jnp.float32),
          pltpu.SemaphoreType.DMA,
          pltpu.SemaphoreType.DMA,
          pltpu.SemaphoreType.DMA,
          pltpu.SemaphoreType.DMA,
      ],
  )
  def seg_sum(x_hbm, b_hbm, sums_hbm, cnts_hbm,
              idx0, idx1, rows0, rows1, acc_v, cacc_v,
              semi0, semi1, semx0, semx1):
    cid = lax.axis_index("c")
    sid = lax.axis_index("s")
    wid = sid * NC + cid

    zeros = jnp.zeros((LANES,), jnp.float32)
    ones = jnp.ones((LANES,), jnp.float32)
    lane_iota = lax.iota(jnp.int32, LANES)

    def zbody(i, carry):
      for j in range(D // LANES):
        acc_v[pl.ds(i * D + j * LANES, LANES)] = zeros
      cacc_v[pl.ds(i * LANES, LANES)] = zeros
      return carry

    lax.fori_loop(0, NSEG, zbody, 0)

    start = wid * Q
    end = jnp.minimum(start + Q, NROWS)

    idx_b = [idx0, idx1]
    rows_b = [rows0, rows1]
    semi = [semi0, semi1]
    semx = [semx0, semx1]

    def wstart(i):
      return jnp.minimum(start + i * C, NROWS - C)

    def issue(i, b):
      ws = wstart(i)
      pltpu.async_copy(b_hbm.at[pl.ds(ws, C)], idx_b[b], semi[b])
      pltpu.async_copy(x_hbm.at[pl.ds(ws, C)], rows_b[b], semx[b])

    def wait(i, b):
      ws = wstart(i)
      pltpu.make_async_copy(b_hbm.at[pl.ds(ws, C)], idx_b[b], semi[b]).wait()
      pltpu.make_async_copy(x_hbm.at[pl.ds(ws, C)], rows_b[b], semx[b]).wait()

    def process(i, b):
      lo = start + i * C
      ws = wstart(i)
      ib = idx_b[b]
      rb = rows_b[b]

      ids_first = ib[pl.ds(0, LANES)]
      ids_last = ib[pl.ds(C - LANES, LANES)]
      wuni = ((ids_first[0] == ids_last[LANES - 1])
              & (ws >= lo) & (ws + C <= end))

      @pl.when(wuni)
      def _window_uniform():
        for g in range(G):
          ids16 = ib[pl.ds(g * LANES, LANES)]
          plsc.addupdate_scatter(cacc_v, [ids16 * LANES + lane_iota], ones)
        addr = lax.broadcast(ids_first[0] * D, (LANES,)) + lane_iota

        @plsc.parallel_loop(0, D // LANES, unroll=2)
        def _jbody(j):
          parts = []
          for g in range(G):
            parts.append(_tree_sum(
                [rb[g * LANES + l, pl.ds(j * LANES, LANES)]
                 for l in range(LANES)]))
          plsc.addupdate_scatter(acc_v, [addr + j * LANES], _tree_sum(parts))

      def gbody(g, carry):
        r0 = ws + g * LANES
        ids16 = ib[pl.ds(g * LANES, LANES)]
        gr = lax.broadcast(r0, (LANES,)) + lane_iota
        vmask = (gr >= lo) & (gr < end)
        plsc.addupdate_scatter(
            cacc_v, [ids16 * LANES + lane_iota], ones, mask=vmask)

        full = (ids16[0] == ids16[LANES - 1]) & (r0 >= lo) & (r0 + LANES <= end)

        @pl.when(full)
        def _fast():
          addr = lax.broadcast(ids16[0] * D, (LANES,)) + lane_iota
          for j in range(D // LANES):
            s = _tree_sum(
                [rb[g * LANES + l, pl.ds(j * LANES, LANES)]
                 for l in range(LANES)])
            plsc.addupdate_scatter(acc_v, [addr + (j * LANES)], s)

        @pl.when(jnp.logical_not(full))
        def _slow():
          idsD = ids16 * D
          for l in range(LANES):
            rl = r0 + l
            inb = (rl >= lo) & (rl < end)
            m = lax.broadcast(inb, (LANES,))
            seg = lax.broadcast(idsD[l], (LANES,)) + lane_iota

            def sjbody(j, _l=l, _seg=seg, _m=m):
              plsc.addupdate_scatter(
                  acc_v, [_seg + j * LANES],
                  rb[g * LANES + _l, pl.ds(j * LANES, LANES)], mask=_m)

            plsc.parallel_loop(0, D // LANES, unroll=4)(sjbody)

        return carry

      @pl.when(jnp.logical_not(wuni) & (lo < end))
      def _():
        lax.fori_loop(0, G, gbody, 0)

    issue(0, 0)

    def pbody(p, carry):
      w = p * 2
      issue(w + 1, 1)
      wait(w, 0)
      process(w, 0)
      issue(w + 2, 0)
      wait(w + 1, 1)
      process(w + 1, 1)
      return carry

    lax.fori_loop(0, (NWIN - 1) // 2, pbody, 0)
    wait(NWIN - 1, 0)
    process(NWIN - 1, 0)

    pltpu.sync_copy(acc_v, sums_hbm.at[wid])
    pltpu.sync_copy(cacc_v, cnts_hbm.at[wid])

  return seg_sum(x, batch_i32)


def _combine(sums, cnts):
  def body(s_ref, c_ref, o_ref):
    s = jnp.sum(s_ref[...], axis=0)
    c = jnp.sum(c_ref[...], axis=(0, 2))
    o_ref[...] = s / jnp.maximum(c, 1.0)[:, None]

  return pl.pallas_call(
      body,
      out_shape=jax.ShapeDtypeStruct((NSEG, D), jnp.float32),
  )(sums, cnts)


@jax.jit
def kernel(x, batch):
  sums, cnts = _sc_segment_sums(x, batch.astype(jnp.int32))
  sums = sums.reshape(NW, NSEG, D)
  cnts = cnts.reshape(NW, NSEG, LANES)
  return _combine(sums, cnts)


# speedup vs baseline: 1.0807x; 1.0230x over previous
import functools

import jax
import jax.numpy as jnp
from jax import lax
from jax.experimental import pallas as pl
from jax.experimental.pallas import tpu as pltpu
from jax.experimental.pallas import tpu_sc as plsc

NSEG = 128
NROWS = 50000
D = 512
LANES = 16
C = 48
G = C // LANES
NC = 2
NS = 16
NW = NC * NS
Q = (-(-NROWS // NW) + 7) // 8 * 8
NWIN = -(-Q // C)


def _tree_sum(vs):
  while len(vs) > 1:
    nxt = [a + b for a, b in zip(vs[::2], vs[1::2])]
    if len(vs) % 2:
      nxt.append(vs[-1])
    vs = nxt
  return vs[0]


def _sc_segment_sums(x, batch_i32):
  mesh = plsc.VectorSubcoreMesh(core_axis_name="c", subcore_axis_name="s")

  @functools.partial(
      pl.kernel,
      mesh=mesh,
      compiler_params=pltpu.CompilerParams(needs_layout_passes=False),
      out_type=[
          jax.ShapeDtypeStruct((NW, NSEG * D), jnp.float32),
          jax.ShapeDtypeStruct((NW, NSEG * LANES), jnp.float32),
      ],
      scratch_types=[
          pltpu.VMEM((C,), jnp.int32),
          pltpu.VMEM((C,), jnp.int32),
          pltpu.VMEM((C, D), jnp.float32),
          pltpu.VMEM((C, D), jnp.float32),
          pltpu.VMEM((NSEG * D,), jnp.float32),
          pltpu.VMEM((NSEG * LANES,), jnp.float32),
          pltpu.SemaphoreType.DMA,
          pltpu.SemaphoreType.DMA,
          pltpu.SemaphoreType.DMA,
          pltpu.SemaphoreType.DMA,
      ],
  )
  def seg_sum(x_hbm, b_hbm, sums_hbm, cnts_hbm,
              idx0, idx1, rows0, rows1, acc_v, cacc_v,
              semi0, semi1, semx0, semx1):
    cid = lax.axis_index("c")
    sid = lax.axis_index("s")
    wid = sid * NC + cid

    zeros = jnp.zeros((LANES,), jnp.float32)
    ones = jnp.ones((LANES,), jnp.float32)
    lane_iota = lax.iota(jnp.int32, LANES)

    def zbody(i, carry):
      for j in range(D // LANES):
        acc_v[pl.ds(i * D + j * LANES, LANES)] = zeros
      cacc_v[pl.ds(i * LANES, LANES)] = zeros
      return carry

    lax.fori_loop(0, NSEG, zbody, 0)

    start = wid * Q
    end = jnp.minimum(start + Q, NROWS)

    idx_b = [idx0, idx1]
    rows_b = [rows0, rows1]
    semi = [semi0, semi1]
    semx = [semx0, semx1]

    def wstart(i):
      return jnp.minimum(start + i * C, NROWS - C)

    def issue(i, b):
      ws = wstart(i)
      pltpu.async_copy(b_hbm.at[pl.ds(ws, C)], idx_b[b], semi[b])
      pltpu.async_copy(x_hbm.at[pl.ds(ws, C)], rows_b[b], semx[b])

    def wait(i, b):
      ws = wstart(i)
      pltpu.make_async_copy(b_hbm.at[pl.ds(ws, C)], idx_b[b], semi[b]).wait()
      pltpu.make_async_copy(x_hbm.at[pl.ds(ws, C)], rows_b[b], semx[b]).wait()

    def process(i, b):
      lo = start + i * C
      ws = wstart(i)
      ib = idx_b[b]
      rb = rows_b[b]

      ids_first = ib[pl.ds(0, LANES)]
      ids_last = ib[pl.ds(C - LANES, LANES)]
      wuni = ((ids_first[0] == ids_last[LANES - 1])
              & (ws >= lo) & (ws + C <= end))

      @pl.when(wuni)
      def _window_uniform():
        seg0 = ids_first[0]
        coff = seg0 * LANES
        cacc_v[pl.ds(coff, LANES)] = cacc_v[pl.ds(coff, LANES)] + float(G)
        base = seg0 * D

        @plsc.parallel_loop(0, D // LANES, unroll=2)
        def _jbody(j):
          parts = []
          for g in range(G):
            parts.append(_tree_sum(
                [rb[g * LANES + l, pl.ds(j * LANES, LANES)]
                 for l in range(LANES)]))
          off = base + j * LANES
          acc_v[pl.ds(off, LANES)] = acc_v[pl.ds(off, LANES)] + _tree_sum(parts)

      def gbody(g, carry):
        r0 = ws + g * LANES
        ids16 = ib[pl.ds(g * LANES, LANES)]
        gr = lax.broadcast(r0, (LANES,)) + lane_iota
        vmask = (gr >= lo) & (gr < end)
        plsc.addupdate_scatter(
            cacc_v, [ids16 * LANES + lane_iota], ones, mask=vmask)

        full = (ids16[0] == ids16[LANES - 1]) & (r0 >= lo) & (r0 + LANES <= end)

        @pl.when(full)
        def _fast():
          base = ids16[0] * D
          for j in range(D // LANES):
            s = _tree_sum(
                [rb[g * LANES + l, pl.ds(j * LANES, LANES)]
                 for l in range(LANES)])
            off = base + j * LANES
            acc_v[pl.ds(off, LANES)] = acc_v[pl.ds(off, LANES)] + s

        @pl.when(jnp.logical_not(full))
        def _slow():
          idsD = ids16 * D
          for l in range(LANES):
            rl = r0 + l
            inb = (rl >= lo) & (rl < end)
            m = lax.broadcast(inb, (LANES,))
            seg = lax.broadcast(idsD[l], (LANES,)) + lane_iota

            def sjbody(j, _l=l, _seg=seg, _m=m):
              plsc.addupdate_scatter(
                  acc_v, [_seg + j * LANES],
                  rb[g * LANES + _l, pl.ds(j * LANES, LANES)], mask=_m)

            plsc.parallel_loop(0, D // LANES, unroll=4)(sjbody)

        return carry

      @pl.when(jnp.logical_not(wuni) & (lo < end))
      def _():
        lax.fori_loop(0, G, gbody, 0)

    issue(0, 0)

    def pbody(p, carry):
      w = p * 2
      issue(w + 1, 1)
      wait(w, 0)
      process(w, 0)
      issue(w + 2, 0)
      wait(w + 1, 1)
      process(w + 1, 1)
      return carry

    lax.fori_loop(0, (NWIN - 1) // 2, pbody, 0)
    wait(NWIN - 1, 0)
    process(NWIN - 1, 0)

    pltpu.sync_copy(acc_v, sums_hbm.at[wid])
    pltpu.sync_copy(cacc_v, cnts_hbm.at[wid])

  return seg_sum(x, batch_i32)


def _combine(sums, cnts):
  def body(s_ref, c_ref, o_ref):
    s = jnp.sum(s_ref[...], axis=0)
    c = jnp.sum(c_ref[...], axis=(0, 2))
    o_ref[...] = s / jnp.maximum(c, 1.0)[:, None]

  return pl.pallas_call(
      body,
      out_shape=jax.ShapeDtypeStruct((NSEG, D), jnp.float32),
  )(sums, cnts)


@jax.jit
def kernel(x, batch):
  sums, cnts = _sc_segment_sums(x, batch.astype(jnp.int32))
  sums = sums.reshape(NW, NSEG, D)
  cnts = cnts.reshape(NW, NSEG, LANES)
  return _combine(sums, cnts)


# speedup vs baseline: 1.1611x; 1.0744x over previous
import functools

import jax
import jax.numpy as jnp
from jax import lax
from jax.experimental import pallas as pl
from jax.experimental.pallas import tpu as pltpu
from jax.experimental.pallas import tpu_sc as plsc

NSEG = 128
NROWS = 50000
D = 512
LANES = 16
C = 32
G = C // LANES
NB = 3
NC = 2
NS = 16
NW = NC * NS
Q = (-(-NROWS // NW) + 15) // 16 * 16
NWIN = Q // C
NWIN_PAD = -(-NWIN // NB) * NB


def _tree_sum(vs):
  while len(vs) > 1:
    nxt = [a + b for a, b in zip(vs[::2], vs[1::2])]
    if len(vs) % 2:
      nxt.append(vs[-1])
    vs = nxt
  return vs[0]


def _sc_segment_sums(x, batch_i32):
  mesh = plsc.VectorSubcoreMesh(core_axis_name="c", subcore_axis_name="s")

  @functools.partial(
      pl.kernel,
      mesh=mesh,
      compiler_params=pltpu.CompilerParams(needs_layout_passes=False),
      out_type=[
          jax.ShapeDtypeStruct((NW, NSEG * D), jnp.float32),
          jax.ShapeDtypeStruct((NW, NSEG * LANES), jnp.float32),
      ],
      scratch_types=[
          pltpu.VMEM((Q,), jnp.int32),
          pltpu.VMEM((C, D), jnp.float32),
          pltpu.VMEM((C, D), jnp.float32),
          pltpu.VMEM((C, D), jnp.float32),
          pltpu.VMEM((NSEG * D,), jnp.float32),
          pltpu.VMEM((NSEG * LANES,), jnp.float32),
          pltpu.SemaphoreType.DMA,
          pltpu.SemaphoreType.DMA,
          pltpu.SemaphoreType.DMA,
          pltpu.SemaphoreType.DMA,
      ],
  )
  def seg_sum(x_hbm, b_hbm, sums_hbm, cnts_hbm,
              idx_v, rows0, rows1, rows2, acc_v, cacc_v,
              semb, semx0, semx1, semx2):
    cid = lax.axis_index("c")
    sid = lax.axis_index("s")
    wid = sid * NC + cid

    zeros = jnp.zeros((LANES,), jnp.float32)
    ones = jnp.ones((LANES,), jnp.float32)
    lane_iota = lax.iota(jnp.int32, LANES)

    start = wid * Q
    end = jnp.minimum(start + Q, NROWS)
    bstart = jnp.minimum(start, NROWS - Q)

    pltpu.async_copy(b_hbm.at[pl.ds(bstart, Q)], idx_v, semb)

    def zbody(i, carry):
      for j in range(D // LANES):
        acc_v[pl.ds(i * D + j * LANES, LANES)] = zeros
      cacc_v[pl.ds(i * LANES, LANES)] = zeros
      return carry

    lax.fori_loop(0, NSEG, zbody, 0)

    rows_b = [rows0, rows1, rows2]
    semx = [semx0, semx1, semx2]

    def wstart(i):
      return jnp.minimum(start + i * C, NROWS - C)

    def issue(i, b):
      pltpu.async_copy(x_hbm.at[pl.ds(wstart(i), C)], rows_b[b], semx[b])

    def wait(i, b):
      pltpu.make_async_copy(
          x_hbm.at[pl.ds(wstart(i), C)], rows_b[b], semx[b]).wait()

    def process(i, b):
      lo = start + i * C
      ws = wstart(i)
      ip = ws - bstart
      rb = rows_b[b]

      ids_first = idx_v[pl.ds(ip, LANES)]
      ids_last = idx_v[pl.ds(ip + C - LANES, LANES)]
      wuni = ((ids_first[0] == ids_last[LANES - 1])
              & (ws >= lo) & (ws + C <= end))

      @pl.when(wuni)
      def _window_uniform():
        seg0 = ids_first[0]
        coff = seg0 * LANES
        cacc_v[pl.ds(coff, LANES)] = cacc_v[pl.ds(coff, LANES)] + float(G)
        base = seg0 * D

        @plsc.parallel_loop(0, D // LANES, unroll=2)
        def _jbody(j):
          parts = []
          for g in range(G):
            parts.append(_tree_sum(
                [rb[g * LANES + l, pl.ds(j * LANES, LANES)]
                 for l in range(LANES)]))
          off = base + j * LANES
          acc_v[pl.ds(off, LANES)] = acc_v[pl.ds(off, LANES)] + _tree_sum(parts)

      def gbody(g, carry):
        r0 = ws + g * LANES
        ids16 = idx_v[pl.ds(ip + g * LANES, LANES)]
        gr = lax.broadcast(r0, (LANES,)) + lane_iota
        vmask = (gr >= lo) & (gr < end)
        plsc.addupdate_scatter(
            cacc_v, [ids16 * LANES + lane_iota], ones, mask=vmask)

        full = (ids16[0] == ids16[LANES - 1]) & (r0 >= lo) & (r0 + LANES <= end)

        @pl.when(full)
        def _fast():
          base = ids16[0] * D
          for j in range(D // LANES):
            s = _tree_sum(
                [rb[g * LANES + l, pl.ds(j * LANES, LANES)]
                 for l in range(LANES)])
            off = base + j * LANES
            acc_v[pl.ds(off, LANES)] = acc_v[pl.ds(off, LANES)] + s

        @pl.when(jnp.logical_not(full))
        def _slow():
          idsD = ids16 * D
          for l in range(LANES):
            rl = r0 + l
            inb = (rl >= lo) & (rl < end)
            m = lax.broadcast(inb, (LANES,))
            seg = lax.broadcast(idsD[l], (LANES,)) + lane_iota

            def sjbody(j, _l=l, _seg=seg, _m=m):
              plsc.addupdate_scatter(
                  acc_v, [_seg + j * LANES],
                  rb[g * LANES + _l, pl.ds(j * LANES, LANES)], mask=_m)

            plsc.parallel_loop(0, D // LANES, unroll=4)(sjbody)

        return carry

      @pl.when(jnp.logical_not(wuni) & (lo < end))
      def _():
        lax.fori_loop(0, G, gbody, 0)

    for b in range(NB):
      issue(b, b)
    pltpu.make_async_copy(b_hbm.at[pl.ds(bstart, Q)], idx_v, semb).wait()

    def pbody(p, carry):
      w = p * NB
      for q in range(NB):
        wait(w + q, q)
        process(w + q, q)
        issue(w + q + NB, q)
      return carry

    lax.fori_loop(0, NWIN_PAD // NB, pbody, 0)
    for q in range(NB):
      wait(NWIN_PAD + q, q)

    pltpu.sync_copy(acc_v, sums_hbm.at[wid])
    pltpu.sync_copy(cacc_v, cnts_hbm.at[wid])

  return seg_sum(x, batch_i32)


def _combine(sums, cnts):
  def body(s_ref, c_ref, o_ref):
    s = jnp.sum(s_ref[...], axis=0)
    c = jnp.sum(c_ref[...], axis=(0, 2))
    o_ref[...] = s / jnp.maximum(c, 1.0)[:, None]

  return pl.pallas_call(
      body,
      out_shape=jax.ShapeDtypeStruct((NSEG, D), jnp.float32),
  )(sums, cnts)


@jax.jit
def kernel(x, batch):
  sums, cnts = _sc_segment_sums(x, batch.astype(jnp.int32))
  sums = sums.reshape(NW, NSEG, D)
  cnts = cnts.reshape(NW, NSEG, LANES)
  return _combine(sums, cnts)


# speedup vs baseline: 1.1665x; 1.0047x over previous
import functools

import jax
import jax.numpy as jnp
from jax import lax
from jax.experimental import pallas as pl
from jax.experimental.pallas import tpu as pltpu
from jax.experimental.pallas import tpu_sc as plsc

NSEG = 128
NROWS = 50000
D = 512
LANES = 16
C = 32
G = C // LANES
NB = 3
NC = 2
NS = 16
NW = NC * NS

TCR = 1600
TC_NBLK = 16
SC_BASE = TCR * TC_NBLK
SCN = NROWS - SC_BASE
Q = (-(-SCN // NW) + 15) // 16 * 16
NWIN = -(-Q // C)
NWIN_PAD = -(-NWIN // NB) * NB


def _tree_sum(vs):
  while len(vs) > 1:
    nxt = [a + b for a, b in zip(vs[::2], vs[1::2])]
    if len(vs) % 2:
      nxt.append(vs[-1])
    vs = nxt
  return vs[0]


def _sc_segment_sums(x, batch_i32):
  mesh = plsc.VectorSubcoreMesh(core_axis_name="c", subcore_axis_name="s")

  @functools.partial(
      pl.kernel,
      mesh=mesh,
      compiler_params=pltpu.CompilerParams(needs_layout_passes=False),
      out_type=[
          jax.ShapeDtypeStruct((NW, NSEG * D), jnp.float32),
          jax.ShapeDtypeStruct((NW, NSEG * LANES), jnp.float32),
      ],
      scratch_types=[
          pltpu.VMEM((Q,), jnp.int32),
          pltpu.VMEM((C, D), jnp.float32),
          pltpu.VMEM((C, D), jnp.float32),
          pltpu.VMEM((C, D), jnp.float32),
          pltpu.VMEM((NSEG * D,), jnp.float32),
          pltpu.VMEM((NSEG * LANES,), jnp.float32),
          pltpu.SemaphoreType.DMA,
          pltpu.SemaphoreType.DMA,
          pltpu.SemaphoreType.DMA,
          pltpu.SemaphoreType.DMA,
      ],
  )
  def seg_sum(x_hbm, b_hbm, sums_hbm, cnts_hbm,
              idx_v, rows0, rows1, rows2, acc_v, cacc_v,
              semb, semx0, semx1, semx2):
    cid = lax.axis_index("c")
    sid = lax.axis_index("s")
    wid = sid * NC + cid

    zeros = jnp.zeros((LANES,), jnp.float32)
    ones = jnp.ones((LANES,), jnp.float32)
    lane_iota = lax.iota(jnp.int32, LANES)

    start = SC_BASE + wid * Q
    end = jnp.minimum(start + Q, NROWS)
    bstart = jnp.minimum(start, NROWS - Q)

    pltpu.async_copy(b_hbm.at[pl.ds(bstart, Q)], idx_v, semb)

    def zbody(i, carry):
      for j in range(D // LANES):
        acc_v[pl.ds(i * D + j * LANES, LANES)] = zeros
      cacc_v[pl.ds(i * LANES, LANES)] = zeros
      return carry

    lax.fori_loop(0, NSEG, zbody, 0)

    rows_b = [rows0, rows1, rows2]
    semx = [semx0, semx1, semx2]

    def wstart(i):
      return jnp.minimum(start + i * C, NROWS - C)

    def issue(i, b):
      pltpu.async_copy(x_hbm.at[pl.ds(wstart(i), C)], rows_b[b], semx[b])

    def wait(i, b):
      pltpu.make_async_copy(
          x_hbm.at[pl.ds(wstart(i), C)], rows_b[b], semx[b]).wait()

    def process(i, b):
      lo = start + i * C
      ws = wstart(i)
      ip = ws - bstart
      rb = rows_b[b]

      ids_first = idx_v[pl.ds(ip, LANES)]
      ids_last = idx_v[pl.ds(ip + C - LANES, LANES)]
      wuni = ((ids_first[0] == ids_last[LANES - 1])
              & (ws >= lo) & (ws + C <= end))

      @pl.when(wuni)
      def _window_uniform():
        seg0 = ids_first[0]
        coff = seg0 * LANES
        cacc_v[pl.ds(coff, LANES)] = cacc_v[pl.ds(coff, LANES)] + float(G)
        base = seg0 * D

        @plsc.parallel_loop(0, D // LANES, unroll=2)
        def _jbody(j):
          parts = []
          for g in range(G):
            parts.append(_tree_sum(
                [rb[g * LANES + l, pl.ds(j * LANES, LANES)]
                 for l in range(LANES)]))
          off = base + j * LANES
          acc_v[pl.ds(off, LANES)] = acc_v[pl.ds(off, LANES)] + _tree_sum(parts)

      def gbody(g, carry):
        r0 = ws + g * LANES
        ids16 = idx_v[pl.ds(ip + g * LANES, LANES)]
        gr = lax.broadcast(r0, (LANES,)) + lane_iota
        vmask = (gr >= lo) & (gr < end)
        plsc.addupdate_scatter(
            cacc_v, [ids16 * LANES + lane_iota], ones, mask=vmask)

        full = (ids16[0] == ids16[LANES - 1]) & (r0 >= lo) & (r0 + LANES <= end)

        @pl.when(full)
        def _fast():
          base = ids16[0] * D
          for j in range(D // LANES):
            s = _tree_sum(
                [rb[g * LANES + l, pl.ds(j * LANES, LANES)]
                 for l in range(LANES)])
            off = base + j * LANES
            acc_v[pl.ds(off, LANES)] = acc_v[pl.ds(off, LANES)] + s

        @pl.when(jnp.logical_not(full))
        def _slow():
          idsD = ids16 * D
          for l in range(LANES):
            rl = r0 + l
            inb = (rl >= lo) & (rl < end)
            m = lax.broadcast(inb, (LANES,))
            seg = lax.broadcast(idsD[l], (LANES,)) + lane_iota

            def sjbody(j, _l=l, _seg=seg, _m=m):
              plsc.addupdate_scatter(
                  acc_v, [_seg + j * LANES],
                  rb[g * LANES + _l, pl.ds(j * LANES, LANES)], mask=_m)

            plsc.parallel_loop(0, D // LANES, unroll=4)(sjbody)

        return carry

      @pl.when(jnp.logical_not(wuni) & (lo < end))
      def _():
        lax.fori_loop(0, G, gbody, 0)

    for b in range(NB):
      issue(b, b)
    pltpu.make_async_copy(b_hbm.at[pl.ds(bstart, Q)], idx_v, semb).wait()

    def pbody(p, carry):
      w = p * NB
      for q in range(NB):
        wait(w + q, q)
        process(w + q, q)
        issue(w + q + NB, q)
      return carry

    lax.fori_loop(0, NWIN_PAD // NB, pbody, 0)
    for q in range(NB):
      wait(NWIN_PAD + q, q)

    pltpu.sync_copy(acc_v, sums_hbm.at[wid])
    pltpu.sync_copy(cacc_v, cnts_hbm.at[wid])

  return seg_sum(x, batch_i32)


def _tc_partial(x, batch_i32):
  b3 = batch_i32[:SC_BASE].reshape(TC_NBLK, 1, TCR)

  def body(x_ref, b_ref, os_ref, oc_ref):
    i = pl.program_id(0)
    ids = b_ref[0, 0, :]
    oh = (lax.broadcasted_iota(jnp.int32, (NSEG, TCR), 0)
          == ids[None, :]).astype(jnp.float32)
    part = lax.dot_general(
        oh, x_ref[...], dimension_numbers=(((1,), (0,)), ((), ())),
        preferred_element_type=jnp.float32)
    cnt = jnp.sum(oh, axis=1)[None, :]

    @pl.when(i == 0)
    def _():
      os_ref[...] = jnp.zeros_like(os_ref)
      oc_ref[...] = jnp.zeros_like(oc_ref)

    os_ref[...] += part
    oc_ref[...] += cnt

  return pl.pallas_call(
      body,
      grid=(TC_NBLK,),
      in_specs=[
          pl.BlockSpec((TCR, D), lambda i: (i, 0)),
          pl.BlockSpec((1, 1, TCR), lambda i: (i, 0, 0)),
      ],
      out_specs=[
          pl.BlockSpec((NSEG, D), lambda i: (0, 0)),
          pl.BlockSpec((1, NSEG), lambda i: (0, 0)),
      ],
      out_shape=[
          jax.ShapeDtypeStruct((NSEG, D), jnp.float32),
          jax.ShapeDtypeStruct((1, NSEG), jnp.float32),
      ],
  )(x, b3)


def _combine(sums, cnts, tsums, tcnts):
  def body(s_ref, c_ref, ts_ref, tc_ref, o_ref):
    s = jnp.sum(s_ref[...], axis=0) + ts_ref[...]
    c = jnp.sum(c_ref[...], axis=(0, 2)) + tc_ref[0]
    o_ref[...] = s / jnp.maximum(c, 1.0)[:, None]

  return pl.pallas_call(
      body,
      out_shape=jax.ShapeDtypeStruct((NSEG, D), jnp.float32),
  )(sums, cnts, tsums, tcnts)


@jax.jit
def kernel(x, batch):
  batch_i32 = batch.astype(jnp.int32)
  sums, cnts = _sc_segment_sums(x, batch_i32)
  tsums, tcnts = _tc_partial(x, batch_i32)
  sums = sums.reshape(NW, NSEG, D)
  cnts = cnts.reshape(NW, NSEG, LANES)
  return _combine(sums, cnts, tsums, tcnts)
